# baseline (device time: 56047 ns/iter reference)
import jax
import jax.numpy as jnp
from jax import lax
from jax.experimental import pallas as pl
from jax.experimental.pallas import tpu as pltpu

N_DEV = 16
BLK = 32
SQ = 512
HQ = 8
HKV = 2
DH = 128
D = 1024
SCALE = 0.08838834764831843
GQ = HQ // HKV


def kernel(x, Wq, Wo, K_ext, V_ext):
    skv = K_ext.shape[1]

    def body(x_ref, wq_ref, wo_ref, k_ref, v_ref, out_ref,
             sendb_o, sendb_l, recvb_o, recvb_l, gath,
             rso_ss, rso_rs, rsl_ss, rsl_rs, ag_ss, ag_rs):
        my = lax.axis_index("i")
        mybase = pl.multiple_of(my * BLK, BLK)

        barrier_sem = pltpu.get_barrier_semaphore()
        for j in range(N_DEV):
            @pl.when(my != j)
            def _():
                pl.semaphore_signal(
                    barrier_sem, inc=1,
                    device_id=(j,), device_id_type=pl.DeviceIdType.MESH,
                )
        pl.semaphore_wait(barrier_sem, N_DEV - 1)

        xb = x_ref[0].astype(jnp.bfloat16)
        wq = wq_ref[...].astype(jnp.bfloat16)
        q = lax.dot_general(
            xb, wq, (((1,), (0,)), ((), ())),
            preferred_element_type=jnp.float32,
        )
        q = (q * SCALE).astype(jnp.bfloat16)

        k2 = k_ref[0].reshape(skv, HKV * DH).astype(jnp.bfloat16)
        v2 = v_ref[0].reshape(skv, HKV * DH).astype(jnp.bfloat16)

        def rs_desc(j):
            return (
                pltpu.make_async_remote_copy(
                    src_ref=sendb_o.at[pl.ds(j * BLK, BLK)],
                    dst_ref=recvb_o.at[pl.ds(mybase, BLK)],
                    send_sem=rso_ss.at[j],
                    recv_sem=rso_rs.at[my],
                    device_id=(j,),
                    device_id_type=pl.DeviceIdType.MESH,
                ),
                pltpu.make_async_remote_copy(
                    src_ref=sendb_l.at[pl.ds(j * BLK, BLK)],
                    dst_ref=recvb_l.at[pl.ds(mybase, BLK)],
                    send_sem=rsl_ss.at[j],
                    recv_sem=rsl_rs.at[my],
                    device_id=(j,),
                    device_id_type=pl.DeviceIdType.MESH,
                ),
            )

        for half in range(2):
            r0 = half * (SQ // 2)
            o_parts = []
            l_parts = []
            for h in range(HQ):
                g = h // GQ
                qh = q[r0:r0 + SQ // 2, h * DH:(h + 1) * DH]
                kg = k2[:, g * DH:(g + 1) * DH]
                vg = v2[:, g * DH:(g + 1) * DH]
                s = lax.dot_general(
                    qh, kg, (((1,), (1,)), ((), ())),
                    preferred_element_type=jnp.float32,
                )
                p = jnp.exp(s)
                l_parts.append(jnp.sum(p, axis=1, keepdims=True))
                o_parts.append(lax.dot_general(
                    p.astype(jnp.bfloat16), vg, (((1,), (0,)), ((), ())),
                    preferred_element_type=jnp.float32,
                ))
            o_half = jnp.concatenate(o_parts, axis=1)
            l_half = jnp.concatenate(l_parts, axis=1)
            sendb_o[r0:r0 + SQ // 2, :] = o_half.astype(jnp.bfloat16)
            sendb_l[r0:r0 + SQ // 2, :] = l_half
            for j in range(half * (N_DEV // 2), (half + 1) * (N_DEV // 2)):
                @pl.when(my != j)
                def _():
                    d_o, d_l = rs_desc(j)
                    d_o.start()
                    d_l.start()

        recvb_o[pl.ds(mybase, BLK), :] = sendb_o[pl.ds(mybase, BLK), :]
        recvb_l[pl.ds(mybase, BLK), :] = sendb_l[pl.ds(mybase, BLK), :]

        for j in range(N_DEV):
            @pl.when(my != j)
            def _():
                recv_o = pltpu.make_async_remote_copy(
                    src_ref=sendb_o.at[pl.ds(0, BLK)],
                    dst_ref=recvb_o.at[pl.ds(j * BLK, BLK)],
                    send_sem=rso_ss.at[j],
                    recv_sem=rso_rs.at[j],
                    device_id=(j,),
                    device_id_type=pl.DeviceIdType.MESH,
                )
                recv_l = pltpu.make_async_remote_copy(
                    src_ref=sendb_l.at[pl.ds(0, BLK)],
                    dst_ref=recvb_l.at[pl.ds(j * BLK, BLK)],
                    send_sem=rsl_ss.at[j],
                    recv_sem=rsl_rs.at[j],
                    device_id=(j,),
                    device_id_type=pl.DeviceIdType.MESH,
                )
                recv_o.wait_recv()
                recv_l.wait_recv()

        o32 = recvb_o[0:BLK, :].astype(jnp.float32)
        l32 = recvb_l[0:BLK, :]
        for j in range(1, N_DEV):
            o32 = o32 + recvb_o[j * BLK:(j + 1) * BLK, :].astype(jnp.float32)
            l32 = l32 + recvb_l[j * BLK:(j + 1) * BLK, :]

        recip = 1.0 / l32
        on = jnp.concatenate(
            [o32[:, h * DH:(h + 1) * DH] * recip[:, h:h + 1]
             for h in range(HQ)],
            axis=1,
        ).astype(jnp.bfloat16)
        wo = wo_ref[...].astype(jnp.bfloat16)
        y32 = lax.dot_general(
            on, wo, (((1,), (0,)), ((), ())),
            preferred_element_type=jnp.float32,
        )
        gath[pl.ds(mybase, BLK), :] = y32.astype(jnp.bfloat16)

        def ag_desc(j):
            return pltpu.make_async_remote_copy(
                src_ref=gath.at[pl.ds(mybase, BLK)],
                dst_ref=gath.at[pl.ds(mybase, BLK)],
                send_sem=ag_ss.at[j],
                recv_sem=ag_rs.at[my],
                device_id=(j,),
                device_id_type=pl.DeviceIdType.MESH,
            )

        for j in range(N_DEV):
            @pl.when(my != j)
            def _():
                ag_desc(j).start()

        for j in range(N_DEV):
            @pl.when(my != j)
            def _():
                recv = pltpu.make_async_remote_copy(
                    src_ref=gath.at[pl.ds(0, BLK)],
                    dst_ref=gath.at[pl.ds(j * BLK, BLK)],
                    send_sem=ag_ss.at[j],
                    recv_sem=ag_rs.at[j],
                    device_id=(j,),
                    device_id_type=pl.DeviceIdType.MESH,
                )
                recv.wait_recv()

        out_ref[0] = gath[:, :].astype(jnp.float32)

        for j in range(N_DEV):
            @pl.when(my != j)
            def _():
                d_o, d_l = rs_desc(j)
                d_o.wait_send()
                d_l.wait_send()
                ag_desc(j).wait_send()

    return pl.pallas_call(
        body,
        out_shape=jax.ShapeDtypeStruct((1, SQ, D), jnp.float32),
        in_specs=[pl.BlockSpec(memory_space=pltpu.VMEM)] * 5,
        out_specs=pl.BlockSpec(memory_space=pltpu.VMEM),
        scratch_shapes=[
            pltpu.VMEM((SQ, D), jnp.bfloat16),
            pltpu.VMEM((SQ, HQ), jnp.float32),
            pltpu.VMEM((SQ, D), jnp.bfloat16),
            pltpu.VMEM((SQ, HQ), jnp.float32),
            pltpu.VMEM((SQ, D), jnp.bfloat16),
            pltpu.SemaphoreType.DMA((N_DEV,)),
            pltpu.SemaphoreType.DMA((N_DEV,)),
            pltpu.SemaphoreType.DMA((N_DEV,)),
            pltpu.SemaphoreType.DMA((N_DEV,)),
            pltpu.SemaphoreType.DMA((N_DEV,)),
            pltpu.SemaphoreType.DMA((N_DEV,)),
        ],
        compiler_params=pltpu.CompilerParams(collective_id=0),
    )(x, Wq, Wo, K_ext, V_ext)


# device time: 50373 ns/iter; 1.1126x vs baseline; 1.1126x over previous
import jax
import jax.numpy as jnp
from jax import lax
from jax.experimental import pallas as pl
from jax.experimental.pallas import tpu as pltpu

N_DEV = 16
BLK = 32
SQ = 512
HQ = 8
HKV = 2
DH = 128
D = 1024
SCALE = 0.08838834764831843
GQ = HQ // HKV


def kernel(x, Wq, Wo, K_ext, V_ext):
    skv = K_ext.shape[1]

    def body(x_ref, wq_ref, wo_ref, k_ref, v_ref, out_ref,
             sendb_o, sendb_l, recvb_o, recvb_l, gath,
             rso_ss, rso_rs, rso_ss1, rso_rs1,
             rsl_ss, rsl_rs, ag_ss, ag_rs):
        my = lax.axis_index("i")
        mybase = pl.multiple_of(my * BLK, BLK)

        barrier_sem = pltpu.get_barrier_semaphore()
        for j in range(N_DEV):
            @pl.when(my != j)
            def _():
                pl.semaphore_signal(
                    barrier_sem, inc=1,
                    device_id=(j,), device_id_type=pl.DeviceIdType.MESH,
                )
        pl.semaphore_wait(barrier_sem, N_DEV - 1)

        xb = x_ref[0].astype(jnp.bfloat16)
        wq = wq_ref[...].astype(jnp.bfloat16)
        q = lax.dot_general(
            xb, wq, (((1,), (0,)), ((), ())),
            preferred_element_type=jnp.float32,
        )
        q = (q * SCALE).astype(jnp.bfloat16)

        k2 = k_ref[0].reshape(skv, HKV * DH).astype(jnp.bfloat16)
        v2 = v_ref[0].reshape(skv, HKV * DH).astype(jnp.bfloat16)

        HD = D // 2

        def rs_o_desc(j, c, ss, rs):
            return pltpu.make_async_remote_copy(
                src_ref=sendb_o.at[pl.ds(j * BLK, BLK), pl.ds(c * HD, HD)],
                dst_ref=recvb_o.at[pl.ds(mybase, BLK), pl.ds(c * HD, HD)],
                send_sem=ss.at[j],
                recv_sem=rs.at[my],
                device_id=(j,),
                device_id_type=pl.DeviceIdType.MESH,
            )

        def rs_l_desc(j):
            return pltpu.make_async_remote_copy(
                src_ref=sendb_l.at[pl.ds(j * BLK, BLK)],
                dst_ref=recvb_l.at[pl.ds(mybase, BLK)],
                send_sem=rsl_ss.at[j],
                recv_sem=rsl_rs.at[my],
                device_id=(j,),
                device_id_type=pl.DeviceIdType.MESH,
            )

        l_parts = []
        for c, (ss, rs) in enumerate([(rso_ss, rso_rs), (rso_ss1, rso_rs1)]):
            o_parts = []
            for h in range(c * GQ, (c + 1) * GQ):
                qh = q[:, h * DH:(h + 1) * DH]
                kg = k2[:, c * DH:(c + 1) * DH]
                vg = v2[:, c * DH:(c + 1) * DH]
                s = lax.dot_general(
                    qh, kg, (((1,), (1,)), ((), ())),
                    preferred_element_type=jnp.float32,
                )
                p = jnp.exp(s)
                l_parts.append(jnp.sum(p, axis=1, keepdims=True))
                o_parts.append(lax.dot_general(
                    p.astype(jnp.bfloat16), vg, (((1,), (0,)), ((), ())),
                    preferred_element_type=jnp.float32,
                ))
            o_chunk = jnp.concatenate(o_parts, axis=1)
            sendb_o[:, c * HD:(c + 1) * HD] = o_chunk.astype(jnp.bfloat16)
            for j in range(N_DEV):
                @pl.when(my != j)
                def _():
                    rs_o_desc(j, c, ss, rs).start()
        sendb_l[:, :] = jnp.concatenate(l_parts, axis=1)
        for j in range(N_DEV):
            @pl.when(my != j)
            def _():
                rs_l_desc(j).start()

        recvb_o[pl.ds(mybase, BLK), :] = sendb_o[pl.ds(mybase, BLK), :]
        recvb_l[pl.ds(mybase, BLK), :] = sendb_l[pl.ds(mybase, BLK), :]

        for j in range(N_DEV):
            @pl.when(my != j)
            def _():
                for c, (ss, rs) in enumerate(
                    [(rso_ss, rso_rs), (rso_ss1, rso_rs1)]
                ):
                    recv_o = pltpu.make_async_remote_copy(
                        src_ref=sendb_o.at[pl.ds(0, BLK), pl.ds(c * HD, HD)],
                        dst_ref=recvb_o.at[
                            pl.ds(j * BLK, BLK), pl.ds(c * HD, HD)
                        ],
                        send_sem=ss.at[j],
                        recv_sem=rs.at[j],
                        device_id=(j,),
                        device_id_type=pl.DeviceIdType.MESH,
                    )
                    recv_o.wait_recv()
                recv_l = pltpu.make_async_remote_copy(
                    src_ref=sendb_l.at[pl.ds(0, BLK)],
                    dst_ref=recvb_l.at[pl.ds(j * BLK, BLK)],
                    send_sem=rsl_ss.at[j],
                    recv_sem=rsl_rs.at[j],
                    device_id=(j,),
                    device_id_type=pl.DeviceIdType.MESH,
                )
                recv_l.wait_recv()

        o32 = recvb_o[0:BLK, :].astype(jnp.float32)
        l32 = recvb_l[0:BLK, :]
        for j in range(1, N_DEV):
            o32 = o32 + recvb_o[j * BLK:(j + 1) * BLK, :].astype(jnp.float32)
            l32 = l32 + recvb_l[j * BLK:(j + 1) * BLK, :]

        recip = 1.0 / l32
        on = jnp.concatenate(
            [o32[:, h * DH:(h + 1) * DH] * recip[:, h:h + 1]
             for h in range(HQ)],
            axis=1,
        ).astype(jnp.bfloat16)
        wo = wo_ref[...].astype(jnp.bfloat16)
        y32 = lax.dot_general(
            on, wo, (((1,), (0,)), ((), ())),
            preferred_element_type=jnp.float32,
        )
        gath[pl.ds(mybase, BLK), :] = y32.astype(jnp.bfloat16)

        def ag_desc(j):
            return pltpu.make_async_remote_copy(
                src_ref=gath.at[pl.ds(mybase, BLK)],
                dst_ref=gath.at[pl.ds(mybase, BLK)],
                send_sem=ag_ss.at[j],
                recv_sem=ag_rs.at[my],
                device_id=(j,),
                device_id_type=pl.DeviceIdType.MESH,
            )

        for j in range(N_DEV):
            @pl.when(my != j)
            def _():
                ag_desc(j).start()

        for j in range(N_DEV):
            @pl.when(my != j)
            def _():
                recv = pltpu.make_async_remote_copy(
                    src_ref=gath.at[pl.ds(0, BLK)],
                    dst_ref=gath.at[pl.ds(j * BLK, BLK)],
                    send_sem=ag_ss.at[j],
                    recv_sem=ag_rs.at[j],
                    device_id=(j,),
                    device_id_type=pl.DeviceIdType.MESH,
                )
                recv.wait_recv()

        out_ref[0] = gath[:, :].astype(jnp.float32)

        for j in range(N_DEV):
            @pl.when(my != j)
            def _():
                rs_o_desc(j, 0, rso_ss, rso_rs).wait_send()
                rs_o_desc(j, 1, rso_ss1, rso_rs1).wait_send()
                rs_l_desc(j).wait_send()
                ag_desc(j).wait_send()

    return pl.pallas_call(
        body,
        out_shape=jax.ShapeDtypeStruct((1, SQ, D), jnp.float32),
        in_specs=[pl.BlockSpec(memory_space=pltpu.VMEM)] * 5,
        out_specs=pl.BlockSpec(memory_space=pltpu.VMEM),
        scratch_shapes=[
            pltpu.VMEM((SQ, D), jnp.bfloat16),
            pltpu.VMEM((SQ, HQ), jnp.float32),
            pltpu.VMEM((SQ, D), jnp.bfloat16),
            pltpu.VMEM((SQ, HQ), jnp.float32),
            pltpu.VMEM((SQ, D), jnp.bfloat16),
            pltpu.SemaphoreType.DMA((N_DEV,)),
            pltpu.SemaphoreType.DMA((N_DEV,)),
            pltpu.SemaphoreType.DMA((N_DEV,)),
            pltpu.SemaphoreType.DMA((N_DEV,)),
            pltpu.SemaphoreType.DMA((N_DEV,)),
            pltpu.SemaphoreType.DMA((N_DEV,)),
            pltpu.SemaphoreType.DMA((N_DEV,)),
            pltpu.SemaphoreType.DMA((N_DEV,)),
        ],
        compiler_params=pltpu.CompilerParams(collective_id=0),
    )(x, Wq, Wo, K_ext, V_ext)


# device time: 49606 ns/iter; 1.1298x vs baseline; 1.0155x over previous
import jax
import jax.numpy as jnp
from jax import lax
from jax.experimental import pallas as pl
from jax.experimental.pallas import tpu as pltpu

N_DEV = 16
BLK = 32
SQ = 512
HQ = 8
HKV = 2
DH = 128
D = 1024
SCALE = 0.08838834764831843
GQ = HQ // HKV


def kernel(x, Wq, Wo, K_ext, V_ext):
    skv = K_ext.shape[1]

    def body(x_ref, wq_ref, wo_ref, k_ref, v_ref, out_ref,
             sendb_o, sendb_l, recvb_o, recvb_l, gath,
             rso_ss, rso_rs, rsl_ss, rsl_rs, ag_ss, ag_rs):
        my = lax.axis_index("i")
        mybase = pl.multiple_of(my * BLK, BLK)

        barrier_sem = pltpu.get_barrier_semaphore()
        for j in range(N_DEV):
            @pl.when(my != j)
            def _():
                pl.semaphore_signal(
                    barrier_sem, inc=1,
                    device_id=(j,), device_id_type=pl.DeviceIdType.MESH,
                )
        pl.semaphore_wait(barrier_sem, N_DEV - 1)

        xb = x_ref[0].astype(jnp.bfloat16)
        wq = wq_ref[...].astype(jnp.bfloat16)
        q = lax.dot_general(
            xb, wq, (((1,), (0,)), ((), ())),
            preferred_element_type=jnp.float32,
        )
        q = (q * SCALE).astype(jnp.bfloat16)

        k2 = k_ref[0].reshape(skv, HKV * DH).astype(jnp.bfloat16)
        v2 = v_ref[0].reshape(skv, HKV * DH).astype(jnp.bfloat16)

        def rs_o_desc(j, h):
            return pltpu.make_async_remote_copy(
                src_ref=sendb_o.at[pl.ds(j * BLK, BLK), pl.ds(h * DH, DH)],
                dst_ref=recvb_o.at[pl.ds(mybase, BLK), pl.ds(h * DH, DH)],
                send_sem=rso_ss.at[h, j],
                recv_sem=rso_rs.at[h, my],
                device_id=(j,),
                device_id_type=pl.DeviceIdType.MESH,
            )

        def rs_l_desc(j):
            return pltpu.make_async_remote_copy(
                src_ref=sendb_l.at[pl.ds(j * BLK, BLK)],
                dst_ref=recvb_l.at[pl.ds(mybase, BLK)],
                send_sem=rsl_ss.at[j],
                recv_sem=rsl_rs.at[my],
                device_id=(j,),
                device_id_type=pl.DeviceIdType.MESH,
            )

        l_parts = []
        for h in range(HQ):
            g = h // GQ
            qh = q[:, h * DH:(h + 1) * DH]
            kg = k2[:, g * DH:(g + 1) * DH]
            vg = v2[:, g * DH:(g + 1) * DH]
            s = lax.dot_general(
                qh, kg, (((1,), (1,)), ((), ())),
                preferred_element_type=jnp.float32,
            )
            p = jnp.exp(s)
            l_parts.append(jnp.sum(p, axis=1, keepdims=True))
            o_h = lax.dot_general(
                p.astype(jnp.bfloat16), vg, (((1,), (0,)), ((), ())),
                preferred_element_type=jnp.float32,
            )
            sendb_o[:, h * DH:(h + 1) * DH] = o_h.astype(jnp.bfloat16)
            for j in range(N_DEV):
                @pl.when(my != j)
                def _():
                    rs_o_desc(j, h).start()
        sendb_l[:, :] = jnp.concatenate(l_parts, axis=1)
        for j in range(N_DEV):
            @pl.when(my != j)
            def _():
                rs_l_desc(j).start()

        recvb_o[pl.ds(mybase, BLK), :] = sendb_o[pl.ds(mybase, BLK), :]
        recvb_l[pl.ds(mybase, BLK), :] = sendb_l[pl.ds(mybase, BLK), :]

        for j in range(N_DEV):
            @pl.when(my != j)
            def _():
                for h in range(HQ):
                    recv_o = pltpu.make_async_remote_copy(
                        src_ref=sendb_o.at[pl.ds(0, BLK), pl.ds(h * DH, DH)],
                        dst_ref=recvb_o.at[
                            pl.ds(j * BLK, BLK), pl.ds(h * DH, DH)
                        ],
                        send_sem=rso_ss.at[h, j],
                        recv_sem=rso_rs.at[h, j],
                        device_id=(j,),
                        device_id_type=pl.DeviceIdType.MESH,
                    )
                    recv_o.wait_recv()
                recv_l = pltpu.make_async_remote_copy(
                    src_ref=sendb_l.at[pl.ds(0, BLK)],
                    dst_ref=recvb_l.at[pl.ds(j * BLK, BLK)],
                    send_sem=rsl_ss.at[j],
                    recv_sem=rsl_rs.at[j],
                    device_id=(j,),
                    device_id_type=pl.DeviceIdType.MESH,
                )
                recv_l.wait_recv()

        o32 = recvb_o[0:BLK, :].astype(jnp.float32)
        l32 = recvb_l[0:BLK, :]
        for j in range(1, N_DEV):
            o32 = o32 + recvb_o[j * BLK:(j + 1) * BLK, :].astype(jnp.float32)
            l32 = l32 + recvb_l[j * BLK:(j + 1) * BLK, :]

        recip = 1.0 / l32
        on = jnp.concatenate(
            [o32[:, h * DH:(h + 1) * DH] * recip[:, h:h + 1]
             for h in range(HQ)],
            axis=1,
        ).astype(jnp.bfloat16)
        wo = wo_ref[...].astype(jnp.bfloat16)
        y32 = lax.dot_general(
            on, wo, (((1,), (0,)), ((), ())),
            preferred_element_type=jnp.float32,
        )
        gath[pl.ds(mybase, BLK), :] = y32.astype(jnp.bfloat16)

        def ag_desc(j):
            return pltpu.make_async_remote_copy(
                src_ref=gath.at[pl.ds(mybase, BLK)],
                dst_ref=gath.at[pl.ds(mybase, BLK)],
                send_sem=ag_ss.at[j],
                recv_sem=ag_rs.at[my],
                device_id=(j,),
                device_id_type=pl.DeviceIdType.MESH,
            )

        for j in range(N_DEV):
            @pl.when(my != j)
            def _():
                ag_desc(j).start()

        for j in range(N_DEV):
            @pl.when(my != j)
            def _():
                recv = pltpu.make_async_remote_copy(
                    src_ref=gath.at[pl.ds(0, BLK)],
                    dst_ref=gath.at[pl.ds(j * BLK, BLK)],
                    send_sem=ag_ss.at[j],
                    recv_sem=ag_rs.at[j],
                    device_id=(j,),
                    device_id_type=pl.DeviceIdType.MESH,
                )
                recv.wait_recv()

        out_ref[0] = gath[:, :].astype(jnp.float32)

        for j in range(N_DEV):
            @pl.when(my != j)
            def _():
                for h in range(HQ):
                    rs_o_desc(j, h).wait_send()
                rs_l_desc(j).wait_send()
                ag_desc(j).wait_send()

    return pl.pallas_call(
        body,
        out_shape=jax.ShapeDtypeStruct((1, SQ, D), jnp.float32),
        in_specs=[pl.BlockSpec(memory_space=pltpu.VMEM)] * 5,
        out_specs=pl.BlockSpec(memory_space=pltpu.VMEM),
        scratch_shapes=[
            pltpu.VMEM((SQ, D), jnp.bfloat16),
            pltpu.VMEM((SQ, HQ), jnp.float32),
            pltpu.VMEM((SQ, D), jnp.bfloat16),
            pltpu.VMEM((SQ, HQ), jnp.float32),
            pltpu.VMEM((SQ, D), jnp.bfloat16),
            pltpu.SemaphoreType.DMA((HQ, N_DEV)),
            pltpu.SemaphoreType.DMA((HQ, N_DEV)),
            pltpu.SemaphoreType.DMA((N_DEV,)),
            pltpu.SemaphoreType.DMA((N_DEV,)),
            pltpu.SemaphoreType.DMA((N_DEV,)),
            pltpu.SemaphoreType.DMA((N_DEV,)),
        ],
        compiler_params=pltpu.CompilerParams(collective_id=0),
    )(x, Wq, Wo, K_ext, V_ext)


# device time: 48253 ns/iter; 1.1615x vs baseline; 1.0280x over previous
import jax
import jax.numpy as jnp
from jax import lax
from jax.experimental import pallas as pl
from jax.experimental.pallas import tpu as pltpu

N_DEV = 16
BLK = 32
SQ = 512
HQ = 8
HKV = 2
DH = 128
D = 1024
SCALE = 0.08838834764831843
GQ = HQ // HKV


def kernel(x, Wq, Wo, K_ext, V_ext):
    skv = K_ext.shape[1]

    def body(x_ref, wq_ref, wo_ref, k_ref, v_ref, out_ref,
             sendb_o, sendb_l, recvb_o, recvb_l, gath,
             rso_ss, rso_rs, rsl_ss, rsl_rs, ag_ss, ag_rs,
             agm_ss, agm_rs, agb_ss, agb_rs):
        my = lax.axis_index("i")
        mybase = pl.multiple_of(my * BLK, BLK)

        barrier_sem = pltpu.get_barrier_semaphore()
        for j in range(N_DEV):
            @pl.when(my != j)
            def _():
                pl.semaphore_signal(
                    barrier_sem, inc=1,
                    device_id=(j,), device_id_type=pl.DeviceIdType.MESH,
                )
        pl.semaphore_wait(barrier_sem, N_DEV - 1)

        xb = x_ref[0].astype(jnp.bfloat16)
        wq = wq_ref[...].astype(jnp.bfloat16)
        q = lax.dot_general(
            xb, wq, (((1,), (0,)), ((), ())),
            preferred_element_type=jnp.float32,
        )
        q = (q * SCALE).astype(jnp.bfloat16)

        k2 = k_ref[0].reshape(skv, HKV * DH).astype(jnp.bfloat16)
        v2 = v_ref[0].reshape(skv, HKV * DH).astype(jnp.bfloat16)

        def rs_o_desc(j, h):
            return pltpu.make_async_remote_copy(
                src_ref=sendb_o.at[pl.ds(j * BLK, BLK), pl.ds(h * DH, DH)],
                dst_ref=recvb_o.at[pl.ds(mybase, BLK), pl.ds(h * DH, DH)],
                send_sem=rso_ss.at[h, j],
                recv_sem=rso_rs.at[h, my],
                device_id=(j,),
                device_id_type=pl.DeviceIdType.MESH,
            )

        def rs_l_desc(j):
            return pltpu.make_async_remote_copy(
                src_ref=sendb_l.at[pl.ds(j * BLK, BLK)],
                dst_ref=recvb_l.at[pl.ds(mybase, BLK)],
                send_sem=rsl_ss.at[j],
                recv_sem=rsl_rs.at[my],
                device_id=(j,),
                device_id_type=pl.DeviceIdType.MESH,
            )

        l_parts = []
        for h in range(HQ):
            g = h // GQ
            qh = q[:, h * DH:(h + 1) * DH]
            kg = k2[:, g * DH:(g + 1) * DH]
            vg = v2[:, g * DH:(g + 1) * DH]
            s = lax.dot_general(
                qh, kg, (((1,), (1,)), ((), ())),
                preferred_element_type=jnp.float32,
            )
            p = jnp.exp(s)
            l_parts.append(jnp.sum(p, axis=1, keepdims=True))
            o_h = lax.dot_general(
                p.astype(jnp.bfloat16), vg, (((1,), (0,)), ((), ())),
                preferred_element_type=jnp.float32,
            )
            sendb_o[:, h * DH:(h + 1) * DH] = o_h.astype(jnp.bfloat16)
            for j in range(N_DEV):
                @pl.when(my != j)
                def _():
                    rs_o_desc(j, h).start()
        sendb_l[:, :] = jnp.concatenate(l_parts, axis=1)
        for j in range(N_DEV):
            @pl.when(my != j)
            def _():
                rs_l_desc(j).start()

        recvb_o[pl.ds(mybase, BLK), :] = sendb_o[pl.ds(mybase, BLK), :]
        recvb_l[pl.ds(mybase, BLK), :] = sendb_l[pl.ds(mybase, BLK), :]

        for j in range(N_DEV):
            @pl.when(my != j)
            def _():
                for h in range(HQ):
                    recv_o = pltpu.make_async_remote_copy(
                        src_ref=sendb_o.at[pl.ds(0, BLK), pl.ds(h * DH, DH)],
                        dst_ref=recvb_o.at[
                            pl.ds(j * BLK, BLK), pl.ds(h * DH, DH)
                        ],
                        send_sem=rso_ss.at[h, j],
                        recv_sem=rso_rs.at[h, j],
                        device_id=(j,),
                        device_id_type=pl.DeviceIdType.MESH,
                    )
                    recv_o.wait_recv()
                recv_l = pltpu.make_async_remote_copy(
                    src_ref=sendb_l.at[pl.ds(0, BLK)],
                    dst_ref=recvb_l.at[pl.ds(j * BLK, BLK)],
                    send_sem=rsl_ss.at[j],
                    recv_sem=rsl_rs.at[j],
                    device_id=(j,),
                    device_id_type=pl.DeviceIdType.MESH,
                )
                recv_l.wait_recv()

        o32 = recvb_o[0:BLK, :].astype(jnp.float32)
        l32 = recvb_l[0:BLK, :]
        for j in range(1, N_DEV):
            o32 = o32 + recvb_o[j * BLK:(j + 1) * BLK, :].astype(jnp.float32)
            l32 = l32 + recvb_l[j * BLK:(j + 1) * BLK, :]

        recip = 1.0 / l32
        on = jnp.concatenate(
            [o32[:, h * DH:(h + 1) * DH] * recip[:, h:h + 1]
             for h in range(HQ)],
            axis=1,
        ).astype(jnp.bfloat16)
        wo = wo_ref[...].astype(jnp.bfloat16)
        y32 = lax.dot_general(
            on, wo, (((1,), (0,)), ((), ())),
            preferred_element_type=jnp.float32,
        )
        gath[pl.ds(mybase, BLK), :] = y32.astype(jnp.bfloat16)

        def ag_desc(j):
            return pltpu.make_async_remote_copy(
                src_ref=gath.at[pl.ds(mybase, BLK)],
                dst_ref=gath.at[pl.ds(mybase, BLK)],
                send_sem=ag_ss.at[j],
                recv_sem=ag_rs.at[my],
                device_id=(j,),
                device_id_type=pl.DeviceIdType.MESH,
            )

        mir = my ^ 8
        mirbase = pl.multiple_of(mir * BLK, BLK)
        myhalf = my >> 3

        def ag_mir_desc():
            return pltpu.make_async_remote_copy(
                src_ref=gath.at[pl.ds(mybase, BLK)],
                dst_ref=gath.at[pl.ds(mybase, BLK)],
                send_sem=agm_ss.at[0],
                recv_sem=agm_rs.at[0],
                device_id=(mir,),
                device_id_type=pl.DeviceIdType.MESH,
            )

        def ag_fwd_desc(j):
            return pltpu.make_async_remote_copy(
                src_ref=gath.at[pl.ds(mirbase, BLK)],
                dst_ref=gath.at[pl.ds(mirbase, BLK)],
                send_sem=agb_ss.at[j],
                recv_sem=agb_rs.at[my],
                device_id=(j,),
                device_id_type=pl.DeviceIdType.MESH,
            )

        ag_mir_desc().start()
        for j in range(N_DEV):
            @pl.when((my != j) & ((j >> 3) == myhalf))
            def _():
                ag_desc(j).start()

        mir_recv = pltpu.make_async_remote_copy(
            src_ref=gath.at[pl.ds(0, BLK)],
            dst_ref=gath.at[pl.ds(mirbase, BLK)],
            send_sem=agm_ss.at[0],
            recv_sem=agm_rs.at[0],
            device_id=(mir,),
            device_id_type=pl.DeviceIdType.MESH,
        )
        mir_recv.wait_recv()
        for j in range(N_DEV):
            @pl.when((my != j) & ((j >> 3) == myhalf))
            def _():
                ag_fwd_desc(j).start()

        for j in range(N_DEV):
            @pl.when((my != j) & ((j >> 3) == myhalf))
            def _():
                recv_a = pltpu.make_async_remote_copy(
                    src_ref=gath.at[pl.ds(0, BLK)],
                    dst_ref=gath.at[pl.ds(j * BLK, BLK)],
                    send_sem=ag_ss.at[j],
                    recv_sem=ag_rs.at[j],
                    device_id=(j,),
                    device_id_type=pl.DeviceIdType.MESH,
                )
                recv_a.wait_recv()
                recv_b = pltpu.make_async_remote_copy(
                    src_ref=gath.at[pl.ds(0, BLK)],
                    dst_ref=gath.at[pl.ds((j ^ 8) * BLK, BLK)],
                    send_sem=agb_ss.at[j],
                    recv_sem=agb_rs.at[j],
                    device_id=(j,),
                    device_id_type=pl.DeviceIdType.MESH,
                )
                recv_b.wait_recv()

        out_ref[0] = gath[:, :].astype(jnp.float32)

        for j in range(N_DEV):
            @pl.when(my != j)
            def _():
                for h in range(HQ):
                    rs_o_desc(j, h).wait_send()
                rs_l_desc(j).wait_send()

            @pl.when((my != j) & ((j >> 3) == myhalf))
            def _():
                ag_desc(j).wait_send()
                ag_fwd_desc(j).wait_send()
        ag_mir_desc().wait_send()

    return pl.pallas_call(
        body,
        out_shape=jax.ShapeDtypeStruct((1, SQ, D), jnp.float32),
        in_specs=[pl.BlockSpec(memory_space=pltpu.VMEM)] * 5,
        out_specs=pl.BlockSpec(memory_space=pltpu.VMEM),
        scratch_shapes=[
            pltpu.VMEM((SQ, D), jnp.bfloat16),
            pltpu.VMEM((SQ, HQ), jnp.float32),
            pltpu.VMEM((SQ, D), jnp.bfloat16),
            pltpu.VMEM((SQ, HQ), jnp.float32),
            pltpu.VMEM((SQ, D), jnp.bfloat16),
            pltpu.SemaphoreType.DMA((HQ, N_DEV)),
            pltpu.SemaphoreType.DMA((HQ, N_DEV)),
            pltpu.SemaphoreType.DMA((N_DEV,)),
            pltpu.SemaphoreType.DMA((N_DEV,)),
            pltpu.SemaphoreType.DMA((N_DEV,)),
            pltpu.SemaphoreType.DMA((N_DEV,)),
            pltpu.SemaphoreType.DMA((1,)),
            pltpu.SemaphoreType.DMA((1,)),
            pltpu.SemaphoreType.DMA((N_DEV,)),
            pltpu.SemaphoreType.DMA((N_DEV,)),
        ],
        compiler_params=pltpu.CompilerParams(collective_id=0),
    )(x, Wq, Wo, K_ext, V_ext)
